# baseline (speedup 1.0000x reference)
import functools

import jax
import jax.numpy as jnp
from jax import lax
from jax.experimental import pallas as pl
from jax.experimental.pallas import tpu as pltpu
from jax.experimental.pallas import tpu_sc as plsc

C, Dv, Hv, Wv = 3, 128, 128, 128
S = 4 * 64 * 64 * 64
R = 64 * 64 * 64
NW = 32
SPW = S // NW
M = 512
NBLK = SPW // M
NG = M // 16
V = Dv * Hv * Wv * 4 // 16


def _body(gx_hbm, gy_hbm, gz_hbm, tab_hbm, out_hbm, *refs):
    cxr, cyr, czr, fxr, fyr, fzr, posr = refs[:7]
    idxr = refs[7:23]
    gath = refs[23]
    obr = refs[24:27]
    sem = refs[27]
    wid = lax.axis_index("s") * 2 + lax.axis_index("c")
    n = wid // 8
    r0 = (wid % 8) * SPW
    lane = lax.iota(jnp.int32, 16)

    def block(b, _):
        base = wid * SPW + b * M
        pltpu.sync_copy(gx_hbm.at[pl.ds(base, M)], cxr)
        pltpu.sync_copy(gy_hbm.at[pl.ds(base, M)], cyr)
        pltpu.sync_copy(gz_hbm.at[pl.ds(base, M)], czr)

        for g in range(NG):
            s0 = g * 16
            x = (cxr[pl.ds(s0, 16)] + 1.0) * 0.5 * 127.0
            y = (cyr[pl.ds(s0, 16)] + 1.0) * 0.5 * 127.0
            z = (czr[pl.ds(s0, 16)] + 1.0) * 0.5 * 127.0
            ix = x.astype(jnp.int32)
            iy = y.astype(jnp.int32)
            iz = z.astype(jnp.int32)
            fxr[pl.ds(s0, 16)] = x - ix.astype(jnp.float32)
            fyr[pl.ds(s0, 16)] = y - iy.astype(jnp.float32)
            fzr[pl.ds(s0, 16)] = z - iz.astype(jnp.float32)
            a = iz * 16384 + iy * 128 + ix
            q = a >> 2
            m = a & 3
            selb = m == 3
            posr[pl.ds(s0, 16)] = jnp.where(selb, 1, m)
            qsel = q + jnp.where(selb, V, 0)
            for p in range(4):
                off = 4096 * (p >> 1) + 32 * (p & 1)
                idxr[p * 4 + (g >> 3)][pl.ds((g & 7) * 16, 16)] = qsel + off

        handles = []
        for k in range(16):
            h = pltpu.async_copy(
                tab_hbm.at[idxr[k]], gath.at[pl.ds(k * 128, 128)], sem)
            handles.append(h)
        for h in handles:
            h.wait()

        def pass_b(g, _):
            fx = fxr[pl.ds(g * 16, 16)]
            fy = fyr[pl.ds(g * 16, 16)]
            fz = fzr[pl.ds(g * 16, 16)]
            pos = posr[pl.ds(g * 16, 16)]
            wx = [1.0 - fx, fx]
            wy = [1.0 - fy, fy]
            wz = [1.0 - fz, fz]
            rowg = g * 16 + lane
            colb = pos << 2
            acc = [jnp.zeros((16,), jnp.float32) for _ in range(3)]
            for p in range(4):
                pw = wz[p >> 1] * wy[p & 1]
                rows = rowg + p * M
                for kx in range(2):
                    w = pw * wx[kx]
                    for c in range(3):
                        col = colb + (4 * kx + c)
                        v = plsc.load_gather(gath, [rows, col])
                        acc[c] = acc[c] + v * w
            for c in range(3):
                obr[c][pl.ds(g * 16, 16)] = acc[c]
            return 0

        lax.fori_loop(0, NG, pass_b, 0)

        for c in range(3):
            dst = n * (3 * R) + c * R + r0 + b * M
            pltpu.sync_copy(obr[c], out_hbm.at[pl.ds(dst, M)])
        return 0

    lax.fori_loop(0, NBLK, block, 0)


@jax.jit
def kernel(samplecoords, image):
    sc = samplecoords.reshape(S, 3)
    gx = sc[:, 0].ravel()
    gy = sc[:, 1].ravel()
    gz = sc[:, 2].ravel()
    vol4 = jnp.zeros((Dv, Hv, Wv, 4), jnp.float32)
    vol4 = vol4.at[..., :3].set(jnp.transpose(image[0], (1, 2, 3, 0)))
    flat = vol4.ravel()
    a_tab = flat.reshape(V, 16)
    b_tab = jnp.concatenate([flat[8:], jnp.zeros(8, jnp.float32)]).reshape(V, 16)
    tab = jnp.concatenate([a_tab, b_tab], axis=0)

    run = pl.kernel(
        _body,
        out_type=jax.ShapeDtypeStruct((S * 3,), jnp.float32),
        mesh=plsc.VectorSubcoreMesh(core_axis_name="c", subcore_axis_name="s"),
        compiler_params=pltpu.CompilerParams(
            needs_layout_passes=False, use_tc_tiling_on_sc=False),
        scratch_types=(
            [pltpu.VMEM((M,), jnp.float32)] * 3
            + [pltpu.VMEM((M,), jnp.float32)] * 3
            + [pltpu.VMEM((M,), jnp.int32)]
            + [pltpu.VMEM((128,), jnp.int32)] * 16
            + [pltpu.VMEM((4 * M, 16), jnp.float32)]
            + [pltpu.VMEM((M,), jnp.float32)] * 3
            + [pltpu.SemaphoreType.DMA]
        ),
    )
    out = run(gx, gy, gz, tab)
    return out.reshape(4, 3, 64, 64, 64)


# speedup vs baseline: 2.9956x; 2.9956x over previous
import jax
import jax.numpy as jnp
from jax import lax
from jax.experimental import pallas as pl
from jax.experimental.pallas import tpu as pltpu
from jax.experimental.pallas import tpu_sc as plsc

C, Dv, Hv, Wv = 3, 128, 128, 128
NVOX = Dv * Hv * Wv
S = 4 * 64 * 64 * 64
R = 64 * 64 * 64
NW = 32
SPW = S // NW
M = 512
NBLK = SPW // M
NG = M // 16
V = NVOX * 4 // 16
TF = NVOX * 4

VW = NVOX // NW
VB = 2048
NPB = VW // VB

_SC_PARAMS = pltpu.CompilerParams(
    needs_layout_passes=False, use_tc_tiling_on_sc=False)


def _table_body(c0_hbm, c1_hbm, c2_hbm, tab_hbm, stage, outstage, sem):
    wid = lax.axis_index("s") * 2 + lax.axis_index("c")
    lane = lax.iota(jnp.int32, 16)
    zv = jnp.zeros((16,), jnp.float32)

    def zero_body(i, _):
        stage[pl.ds(3 * VB + i * 16, 16)] = zv
        return 0

    lax.fori_loop(0, VB // 16, zero_body, 0)

    ktpl = (lane & 3) * VB + (lane >> 2)

    def block(b, _):
        v0 = wid * VW + b * VB
        pltpu.sync_copy(c0_hbm.at[pl.ds(v0, VB)], stage.at[pl.ds(0, VB)])
        pltpu.sync_copy(c1_hbm.at[pl.ds(v0, VB)], stage.at[pl.ds(VB, VB)])
        pltpu.sync_copy(c2_hbm.at[pl.ds(v0, VB)], stage.at[pl.ds(2 * VB, VB)])

        def ileave(go, _):
            for gi in range(8):
                g = go * 8 + gi
                v = plsc.load_gather(stage, [ktpl + g * 4])
                outstage[pl.ds(g * 16, 16)] = v
            return 0

        lax.fori_loop(0, VB * 4 // 16 // 8, ileave, 0)

        pltpu.sync_copy(outstage, tab_hbm.at[pl.ds(4 * v0, 4 * VB)])

        @pl.when(jnp.logical_or(wid > 0, b > 0))
        def _():
            pltpu.sync_copy(
                outstage, tab_hbm.at[pl.ds(TF + 4 * v0 - 8, 4 * VB)])

        @pl.when(jnp.logical_and(wid == 0, b == 0))
        def _():
            pltpu.sync_copy(
                outstage.at[pl.ds(8, 4 * VB - 8)],
                tab_hbm.at[pl.ds(TF, 4 * VB - 8)])
        return 0

    lax.fori_loop(0, NPB, block, 0)


def _sample_body(crd_hbm, tab_hbm, out_hbm, *refs):
    cin = refs[0]
    fxr, fyr, fzr, posr = refs[1:5]
    idxr = refs[5:21]
    gath = refs[21]
    obr = refs[22:25]
    sem = refs[25]
    wid = lax.axis_index("s") * 2 + lax.axis_index("c")
    n = wid // 8
    r0 = (wid % 8) * SPW
    lane = lax.iota(jnp.int32, 16)
    lane3 = lane * 3

    def block(b, _):
        base = wid * SPW + b * M
        pltpu.sync_copy(crd_hbm.at[pl.ds(base * 3, M * 3)], cin)

        for g in range(NG):
            s0 = g * 16
            xg = plsc.load_gather(cin, [lane3 + (48 * g + 0)])
            yg = plsc.load_gather(cin, [lane3 + (48 * g + 1)])
            zg = plsc.load_gather(cin, [lane3 + (48 * g + 2)])
            x = (xg + 1.0) * 0.5 * 127.0
            y = (yg + 1.0) * 0.5 * 127.0
            z = (zg + 1.0) * 0.5 * 127.0
            ix = x.astype(jnp.int32)
            iy = y.astype(jnp.int32)
            iz = z.astype(jnp.int32)
            fxr[pl.ds(s0, 16)] = x - ix.astype(jnp.float32)
            fyr[pl.ds(s0, 16)] = y - iy.astype(jnp.float32)
            fzr[pl.ds(s0, 16)] = z - iz.astype(jnp.float32)
            a = iz * 16384 + iy * 128 + ix
            q = a >> 2
            m = a & 3
            selb = m == 3
            posr[pl.ds(s0, 16)] = jnp.where(selb, 1, m)
            qsel = q + jnp.where(selb, V, 0)
            for p in range(4):
                off = 4096 * (p >> 1) + 32 * (p & 1)
                idxr[p * 4 + (g >> 3)][pl.ds((g & 7) * 16, 16)] = qsel + off

        handles = []
        for k in range(16):
            h = pltpu.async_copy(
                tab_hbm.at[idxr[k]], gath.at[pl.ds(k * 128, 128)], sem)
            handles.append(h)
        for h in handles:
            h.wait()

        def pass_b(g, _):
            fx = fxr[pl.ds(g * 16, 16)]
            fy = fyr[pl.ds(g * 16, 16)]
            fz = fzr[pl.ds(g * 16, 16)]
            pos = posr[pl.ds(g * 16, 16)]
            wx = [1.0 - fx, fx]
            wy = [1.0 - fy, fy]
            wz = [1.0 - fz, fz]
            rowg = g * 16 + lane
            colb = pos << 2
            acc = [jnp.zeros((16,), jnp.float32) for _ in range(3)]
            for p in range(4):
                pw = wz[p >> 1] * wy[p & 1]
                rows = rowg + p * M
                for kx in range(2):
                    w = pw * wx[kx]
                    for c in range(3):
                        col = colb + (4 * kx + c)
                        v = plsc.load_gather(gath, [rows, col])
                        acc[c] = acc[c] + v * w
            for c in range(3):
                obr[c][pl.ds(g * 16, 16)] = acc[c]
            return 0

        lax.fori_loop(0, NG, pass_b, 0)

        for c in range(3):
            dst = n * (3 * R) + c * R + r0 + b * M
            pltpu.sync_copy(obr[c], out_hbm.at[pl.ds(dst, M)])
        return 0

    lax.fori_loop(0, NBLK, block, 0)


@jax.jit
def kernel(samplecoords, image):
    crd = samplecoords.reshape(S * 3)
    ch = image.reshape(3, NVOX)

    mesh = plsc.VectorSubcoreMesh(core_axis_name="c", subcore_axis_name="s")

    build_tab = pl.kernel(
        _table_body,
        out_type=jax.ShapeDtypeStruct((2 * TF,), jnp.float32),
        mesh=mesh,
        compiler_params=_SC_PARAMS,
        scratch_types=[
            pltpu.VMEM((4 * VB,), jnp.float32),
            pltpu.VMEM((4 * VB,), jnp.float32),
            pltpu.SemaphoreType.DMA,
        ],
    )
    tab = build_tab(ch[0], ch[1], ch[2]).reshape(2 * V, 16)

    sample = pl.kernel(
        _sample_body,
        out_type=jax.ShapeDtypeStruct((S * 3,), jnp.float32),
        mesh=mesh,
        compiler_params=_SC_PARAMS,
        scratch_types=(
            [pltpu.VMEM((3 * M,), jnp.float32)]
            + [pltpu.VMEM((M,), jnp.float32)] * 3
            + [pltpu.VMEM((M,), jnp.int32)]
            + [pltpu.VMEM((128,), jnp.int32)] * 16
            + [pltpu.VMEM((4 * M, 16), jnp.float32)]
            + [pltpu.VMEM((M,), jnp.float32)] * 3
            + [pltpu.SemaphoreType.DMA]
        ),
    )
    out = sample(crd, tab)
    return out.reshape(4, 3, 64, 64, 64)


# speedup vs baseline: 2.9968x; 1.0004x over previous
import jax
import jax.numpy as jnp
from jax import lax
from jax.experimental import pallas as pl
from jax.experimental.pallas import tpu as pltpu
from jax.experimental.pallas import tpu_sc as plsc

C, Dv, Hv, Wv = 3, 128, 128, 128
NVOX = Dv * Hv * Wv
S = 4 * 64 * 64 * 64
R = 64 * 64 * 64
NW = 32
SPW = S // NW
M = 512
NBLK = SPW // M
NG = M // 16
V = NVOX * 4 // 16
TF = NVOX * 4

VW = NVOX // NW
VB = 2048
NPB = VW // VB

_SC_PARAMS = pltpu.CompilerParams(
    needs_layout_passes=False, use_tc_tiling_on_sc=False)


SVB = VB + 16


def _table_body(c0_hbm, c1_hbm, c2_hbm, tab_hbm, stage, outa, outb, sem):
    wid = lax.axis_index("s") * 2 + lax.axis_index("c")
    lane = lax.iota(jnp.int32, 16)
    zv = jnp.zeros((16,), jnp.float32)

    def zero_body(i, _):
        stage[pl.ds(3 * SVB + i * 16, 16)] = zv
        return 0

    lax.fori_loop(0, SVB // 16, zero_body, 0)

    ktpl = (lane & 3) * SVB + (lane >> 2)

    def block(b, _):
        v0 = wid * VW + b * VB
        last = jnp.logical_and(wid == NW - 1, b == NPB - 1)

        @pl.when(jnp.logical_not(last))
        def _():
            for c, ref in enumerate((c0_hbm, c1_hbm, c2_hbm)):
                pltpu.sync_copy(ref.at[pl.ds(v0, VB + 8)],
                                stage.at[pl.ds(c * SVB, VB + 8)])

        @pl.when(last)
        def _():
            for c, ref in enumerate((c0_hbm, c1_hbm, c2_hbm)):
                pltpu.sync_copy(ref.at[pl.ds(v0, VB)],
                                stage.at[pl.ds(c * SVB, VB)])

        def ileave(go, _):
            for gi in range(8):
                g = go * 8 + gi
                idx = ktpl + g * 4
                outa[g, :] = plsc.load_gather(stage, [idx])
                outb[g, :] = plsc.load_gather(stage, [idx + 2])
            return 0

        lax.fori_loop(0, VB * 4 // 16 // 8, ileave, 0)

        row0 = v0 // 4
        pltpu.sync_copy(outa, tab_hbm.at[pl.ds(row0, 512)])
        pltpu.sync_copy(outb, tab_hbm.at[pl.ds(V + row0, 512)])
        return 0

    lax.fori_loop(0, NPB, block, 0)


def _sample_body(crd_hbm, tab_hbm, out_hbm, *refs):
    cin = refs[0]
    fxr, fyr, fzr, posr = refs[1:5]
    idxr = refs[5:21]
    gath = refs[21]
    obr = refs[22:25]
    sem = refs[25]
    wid = lax.axis_index("s") * 2 + lax.axis_index("c")
    n = wid // 8
    r0 = (wid % 8) * SPW
    lane = lax.iota(jnp.int32, 16)
    lane3 = lane * 3

    def block(b, _):
        base = wid * SPW + b * M
        pltpu.sync_copy(crd_hbm.at[pl.ds(base * 3, M * 3)], cin)

        for g in range(NG):
            s0 = g * 16
            xg = plsc.load_gather(cin, [lane3 + (48 * g + 0)])
            yg = plsc.load_gather(cin, [lane3 + (48 * g + 1)])
            zg = plsc.load_gather(cin, [lane3 + (48 * g + 2)])
            x = (xg + 1.0) * 0.5 * 127.0
            y = (yg + 1.0) * 0.5 * 127.0
            z = (zg + 1.0) * 0.5 * 127.0
            ix = x.astype(jnp.int32)
            iy = y.astype(jnp.int32)
            iz = z.astype(jnp.int32)
            fxr[pl.ds(s0, 16)] = x - ix.astype(jnp.float32)
            fyr[pl.ds(s0, 16)] = y - iy.astype(jnp.float32)
            fzr[pl.ds(s0, 16)] = z - iz.astype(jnp.float32)
            a = iz * 16384 + iy * 128 + ix
            q = a >> 2
            m = a & 3
            selb = m == 3
            posr[pl.ds(s0, 16)] = jnp.where(selb, 1, m)
            qsel = q + jnp.where(selb, V, 0)
            for p in range(4):
                off = 4096 * (p >> 1) + 32 * (p & 1)
                idxr[p * 4 + (g >> 3)][pl.ds((g & 7) * 16, 16)] = qsel + off

        handles = []
        for k in range(16):
            h = pltpu.async_copy(
                tab_hbm.at[idxr[k]], gath.at[pl.ds(k * 128, 128)], sem)
            handles.append(h)
        for h in handles:
            h.wait()

        def pass_b(g, _):
            fx = fxr[pl.ds(g * 16, 16)]
            fy = fyr[pl.ds(g * 16, 16)]
            fz = fzr[pl.ds(g * 16, 16)]
            pos = posr[pl.ds(g * 16, 16)]
            wx = [1.0 - fx, fx]
            wy = [1.0 - fy, fy]
            wz = [1.0 - fz, fz]
            rowg = g * 16 + lane
            colb = pos << 2
            acc = [jnp.zeros((16,), jnp.float32) for _ in range(3)]
            for p in range(4):
                pw = wz[p >> 1] * wy[p & 1]
                rows = rowg + p * M
                for kx in range(2):
                    w = pw * wx[kx]
                    for c in range(3):
                        col = colb + (4 * kx + c)
                        v = plsc.load_gather(gath, [rows, col])
                        acc[c] = acc[c] + v * w
            for c in range(3):
                obr[c][pl.ds(g * 16, 16)] = acc[c]
            return 0

        lax.fori_loop(0, NG, pass_b, 0)

        for c in range(3):
            dst = n * (3 * R) + c * R + r0 + b * M
            pltpu.sync_copy(obr[c], out_hbm.at[pl.ds(dst, M)])
        return 0

    lax.fori_loop(0, NBLK, block, 0)


@jax.jit
def kernel(samplecoords, image):
    crd = samplecoords.reshape(S * 3)
    ch = image.reshape(3, NVOX)

    mesh = plsc.VectorSubcoreMesh(core_axis_name="c", subcore_axis_name="s")

    build_tab = pl.kernel(
        _table_body,
        out_type=jax.ShapeDtypeStruct((2 * V, 16), jnp.float32),
        mesh=mesh,
        compiler_params=_SC_PARAMS,
        scratch_types=[
            pltpu.VMEM((4 * SVB,), jnp.float32),
            pltpu.VMEM((512, 16), jnp.float32),
            pltpu.VMEM((512, 16), jnp.float32),
            pltpu.SemaphoreType.DMA,
        ],
    )
    tab = build_tab(ch[0], ch[1], ch[2])

    sample = pl.kernel(
        _sample_body,
        out_type=jax.ShapeDtypeStruct((S * 3,), jnp.float32),
        mesh=mesh,
        compiler_params=_SC_PARAMS,
        scratch_types=(
            [pltpu.VMEM((3 * M,), jnp.float32)]
            + [pltpu.VMEM((M,), jnp.float32)] * 3
            + [pltpu.VMEM((M,), jnp.int32)]
            + [pltpu.VMEM((128,), jnp.int32)] * 16
            + [pltpu.VMEM((4 * M, 16), jnp.float32)]
            + [pltpu.VMEM((M,), jnp.float32)] * 3
            + [pltpu.SemaphoreType.DMA]
        ),
    )
    out = sample(crd, tab)
    return out.reshape(4, 3, 64, 64, 64)
